# initial kernel scaffold (unmeasured)
import jax
import jax.numpy as jnp
from jax import lax
from jax.experimental import pallas as pl
from jax.experimental.pallas import tpu as pltpu

N_ROWS = 1024
N_COLS = 512
CHUNK = 128
MAX_CHUNKS = N_ROWS // CHUNK


def kernel(x, dest):
    dest2d = dest.reshape(1, N_ROWS)

    def body(x_ref, dest_ref, out_ref, send_buf, send_sems, recv_sems):
        my_x = lax.axis_index("x")
        my_y = lax.axis_index("y")
        my_z = lax.axis_index("z")
        partner = (1 - my_x, my_y, my_z)

        d = dest_ref[:, :]
        km = (d == my_x).astype(jnp.float32)
        sm = 1.0 - km

        i_idx = lax.broadcasted_iota(jnp.int32, (N_ROWS, N_ROWS), 0)
        j_idx = lax.broadcasted_iota(jnp.int32, (N_ROWS, N_ROWS), 1)
        upper = (i_idx < j_idx).astype(jnp.float32)
        kc = jnp.dot(km, upper, preferred_element_type=jnp.float32)
        pos = lax.broadcasted_iota(jnp.float32, (1, N_ROWS), 1)
        sc = pos - kc

        k_count = jnp.sum(km).astype(jnp.int32)
        s_count = N_ROWS - k_count

        off_keep = jnp.where(my_x == 0, 0, s_count)
        off_remote = jnp.where(my_x == 0, 0, k_count)

        kc_i = kc.astype(jnp.int32)
        sc_i = sc.astype(jnp.int32)
        keep_pos = off_keep + kc_i
        pk = ((i_idx == keep_pos) & (km > 0)).astype(jnp.bfloat16)
        ps = ((i_idx == sc_i) & (sm > 0)).astype(jnp.bfloat16)

        xb = x_ref[:, :].astype(jnp.bfloat16)
        out_ref[:, :] = jnp.dot(
            pk, xb, preferred_element_type=jnp.float32
        ).astype(jnp.bfloat16)
        send_buf[:, :] = jnp.dot(
            ps, xb, preferred_element_type=jnp.float32
        ).astype(jnp.bfloat16)

        bsem = pltpu.get_barrier_semaphore()
        pl.semaphore_signal(
            bsem, inc=1, device_id=partner, device_id_type=pl.DeviceIdType.MESH
        )
        pl.semaphore_wait(bsem, 1)

        nch = (s_count + CHUNK - 1) // CHUNK

        def chunk_rdma(j):
            start = jnp.maximum(0, jnp.minimum(j * CHUNK, s_count - CHUNK))
            return pltpu.make_async_remote_copy(
                src_ref=send_buf.at[pl.ds(start, CHUNK), :],
                dst_ref=out_ref.at[pl.ds(off_remote + start, CHUNK), :],
                send_sem=send_sems.at[j],
                recv_sem=recv_sems.at[j],
                device_id=partner,
                device_id_type=pl.DeviceIdType.MESH,
            )

        for j in range(MAX_CHUNKS):
            @pl.when(j < nch)
            def _(j=j):
                chunk_rdma(j).start()

        for j in range(MAX_CHUNKS):
            @pl.when(j < nch)
            def _(j=j):
                rdma = chunk_rdma(j)
                rdma.wait_send()
                rdma.wait_recv()

    return pl.pallas_call(
        body,
        out_shape=jax.ShapeDtypeStruct((N_ROWS, N_COLS), jnp.bfloat16),
        in_specs=[
            pl.BlockSpec(memory_space=pltpu.VMEM),
            pl.BlockSpec(memory_space=pltpu.VMEM),
        ],
        out_specs=pl.BlockSpec(memory_space=pltpu.VMEM),
        scratch_shapes=[
            pltpu.VMEM((N_ROWS, N_COLS), jnp.bfloat16),
            pltpu.SemaphoreType.DMA((MAX_CHUNKS,)),
            pltpu.SemaphoreType.DMA((MAX_CHUNKS,)),
        ],
        compiler_params=pltpu.CompilerParams(collective_id=0),
    )(x, dest2d)


# baseline (device time: 17022 ns/iter reference)
import jax
import jax.numpy as jnp
from jax import lax
from jax.experimental import pallas as pl
from jax.experimental.pallas import tpu as pltpu

N_ROWS = 1024
N_COLS = 512
CHUNK = 128
MAX_CHUNKS = N_ROWS // CHUNK


def kernel(x, dest):
    dest2d = dest.reshape(1, N_ROWS)

    def body(x_ref, dest_ref, out_ref, send_buf, recv_buf, send_sems, recv_sems):
        my_x = lax.axis_index("x")
        my_y = lax.axis_index("y")
        my_z = lax.axis_index("z")
        partner = (1 - my_x, my_y, my_z)

        d = dest_ref[:, :]
        km = (d == my_x).astype(jnp.float32)
        sm = 1.0 - km

        i_idx = lax.broadcasted_iota(jnp.int32, (N_ROWS, N_ROWS), 0)
        j_idx = lax.broadcasted_iota(jnp.int32, (N_ROWS, N_ROWS), 1)
        upper = (i_idx < j_idx).astype(jnp.float32)
        kc = jnp.dot(km, upper, preferred_element_type=jnp.float32)
        pos = lax.broadcasted_iota(jnp.int32, (1, N_ROWS), 1).astype(jnp.float32)
        sc = pos - kc

        k_count = jnp.sum(km).astype(jnp.int32)
        s_count = N_ROWS - k_count

        off_keep = jnp.where(my_x == 0, 0, s_count)
        off_recv = jnp.where(my_x == 0, k_count, 0)

        kc_i = kc.astype(jnp.int32)
        sc_i = sc.astype(jnp.int32)
        keep_pos = off_keep + kc_i
        pk = ((i_idx == keep_pos) & (km > 0)).astype(jnp.bfloat16)
        ps = ((i_idx == sc_i) & (sm > 0)).astype(jnp.bfloat16)

        xb = x_ref[:, :].astype(jnp.bfloat16)
        send_buf[:, :] = jnp.dot(
            ps, xb, preferred_element_type=jnp.float32
        ).astype(jnp.bfloat16)
        out_ref[:, :] = jnp.dot(
            pk, xb, preferred_element_type=jnp.float32
        ).astype(jnp.bfloat16)

        recv_buf[:, :] = jnp.zeros((N_ROWS, N_COLS), jnp.bfloat16)

        bsem = pltpu.get_barrier_semaphore()
        pl.semaphore_signal(
            bsem, inc=1, device_id=partner, device_id_type=pl.DeviceIdType.MESH
        )
        pl.semaphore_wait(bsem, 1)

        nch = (s_count + CHUNK - 1) // CHUNK

        def chunk_rdma(j):
            return pltpu.make_async_remote_copy(
                src_ref=send_buf.at[pl.ds(j * CHUNK, CHUNK), :],
                dst_ref=recv_buf.at[pl.ds(j * CHUNK, CHUNK), :],
                send_sem=send_sems.at[j],
                recv_sem=recv_sems.at[j],
                device_id=partner,
                device_id_type=pl.DeviceIdType.MESH,
            )

        for j in range(MAX_CHUNKS):
            @pl.when(j < nch)
            def _(j=j):
                chunk_rdma(j).start()

        for j in range(MAX_CHUNKS):
            @pl.when(j < nch)
            def _(j=j):
                rdma = chunk_rdma(j)
                rdma.wait_send()
                rdma.wait_recv()

        pr = ((i_idx == off_recv + j_idx) & (j_idx < s_count)).astype(jnp.bfloat16)
        out_ref[:, :] = out_ref[:, :] + jnp.dot(
            pr, recv_buf[:, :], preferred_element_type=jnp.float32
        ).astype(jnp.bfloat16)

    return pl.pallas_call(
        body,
        out_shape=jax.ShapeDtypeStruct((N_ROWS, N_COLS), jnp.bfloat16),
        in_specs=[
            pl.BlockSpec(memory_space=pltpu.VMEM),
            pl.BlockSpec(memory_space=pltpu.VMEM),
        ],
        out_specs=pl.BlockSpec(memory_space=pltpu.VMEM),
        scratch_shapes=[
            pltpu.VMEM((N_ROWS, N_COLS), jnp.bfloat16),
            pltpu.VMEM((N_ROWS, N_COLS), jnp.bfloat16),
            pltpu.SemaphoreType.DMA((MAX_CHUNKS,)),
            pltpu.SemaphoreType.DMA((MAX_CHUNKS,)),
        ],
        compiler_params=pltpu.CompilerParams(collective_id=0),
    )(x, dest2d)


# device time: 13758 ns/iter; 1.2372x vs baseline; 1.2372x over previous
import jax
import jax.numpy as jnp
from jax import lax
from jax.experimental import pallas as pl
from jax.experimental.pallas import tpu as pltpu

N_ROWS = 1024
N_COLS = 512
CHUNK = 128
MAX_CHUNKS = N_ROWS // CHUNK


def kernel(x, dest):
    dest2d = dest.reshape(1, N_ROWS)

    def body(x_ref, dest_ref, out_ref, send_buf, recv_buf, send_sems, recv_sems):
        my_x = lax.axis_index("x")
        my_y = lax.axis_index("y")
        my_z = lax.axis_index("z")
        partner = (1 - my_x, my_y, my_z)

        recv_buf[:, :] = jnp.zeros((N_ROWS, N_COLS), jnp.bfloat16)
        bsem = pltpu.get_barrier_semaphore()
        pl.semaphore_signal(
            bsem, inc=1, device_id=partner, device_id_type=pl.DeviceIdType.MESH
        )

        d = dest_ref[:, :]
        km = (d == my_x).astype(jnp.float32)
        sm = 1.0 - km

        i_idx = lax.broadcasted_iota(jnp.int32, (N_ROWS, N_ROWS), 0)
        j_idx = lax.broadcasted_iota(jnp.int32, (N_ROWS, N_ROWS), 1)
        diff = i_idx - j_idx
        upper = (diff < 0).astype(jnp.float32)
        kc = jnp.dot(km, upper, preferred_element_type=jnp.float32)
        pos = lax.broadcasted_iota(jnp.int32, (1, N_ROWS), 1).astype(jnp.float32)
        sc = pos - kc

        k_count = jnp.sum(km).astype(jnp.int32)
        s_count = N_ROWS - k_count

        off_keep = jnp.where(my_x == 0, 0, s_count)
        off_recv = jnp.where(my_x == 0, k_count, 0)

        kc_i = jnp.where(km > 0, off_keep + kc.astype(jnp.int32), -1)
        sc_i = jnp.where(sm > 0, sc.astype(jnp.int32), -1)

        xb = x_ref[:, :].astype(jnp.bfloat16)
        nch = (s_count + CHUNK - 1) // CHUNK

        pl.semaphore_wait(bsem, 1)

        def chunk_rdma(j):
            return pltpu.make_async_remote_copy(
                src_ref=send_buf.at[pl.ds(j * CHUNK, CHUNK), :],
                dst_ref=recv_buf.at[pl.ds(j * CHUNK, CHUNK), :],
                send_sem=send_sems.at[j],
                recv_sem=recv_sems.at[j],
                device_id=partner,
                device_id_type=pl.DeviceIdType.MESH,
            )

        ii_c = lax.broadcasted_iota(jnp.int32, (CHUNK, N_ROWS), 0)
        for j in range(MAX_CHUNKS):
            @pl.when(j < nch)
            def _(j=j):
                ps_j = (ii_c == sc_i - j * CHUNK).astype(jnp.bfloat16)
                send_buf[pl.ds(j * CHUNK, CHUNK), :] = jnp.dot(
                    ps_j, xb, preferred_element_type=jnp.float32
                ).astype(jnp.bfloat16)
                chunk_rdma(j).start()

        pk = (i_idx == kc_i).astype(jnp.bfloat16)
        out_ref[:, :] = jnp.dot(
            pk, xb, preferred_element_type=jnp.float32
        ).astype(jnp.bfloat16)

        jj_c = lax.broadcasted_iota(jnp.int32, (N_ROWS, CHUNK), 1)
        i_col = lax.broadcasted_iota(jnp.int32, (N_ROWS, CHUNK), 0)
        for j in range(MAX_CHUNKS):
            @pl.when(j < nch)
            def _(j=j):
                chunk_rdma(j).wait_recv()
                pr_j = (i_col == off_recv + j * CHUNK + jj_c).astype(jnp.bfloat16)
                out_ref[:, :] = out_ref[:, :] + jnp.dot(
                    pr_j,
                    recv_buf[pl.ds(j * CHUNK, CHUNK), :],
                    preferred_element_type=jnp.float32,
                ).astype(jnp.bfloat16)

        for j in range(MAX_CHUNKS):
            @pl.when(j < nch)
            def _(j=j):
                chunk_rdma(j).wait_send()

    return pl.pallas_call(
        body,
        out_shape=jax.ShapeDtypeStruct((N_ROWS, N_COLS), jnp.bfloat16),
        in_specs=[
            pl.BlockSpec(memory_space=pltpu.VMEM),
            pl.BlockSpec(memory_space=pltpu.VMEM),
        ],
        out_specs=pl.BlockSpec(memory_space=pltpu.VMEM),
        scratch_shapes=[
            pltpu.VMEM((N_ROWS, N_COLS), jnp.bfloat16),
            pltpu.VMEM((N_ROWS, N_COLS), jnp.bfloat16),
            pltpu.SemaphoreType.DMA((MAX_CHUNKS,)),
            pltpu.SemaphoreType.DMA((MAX_CHUNKS,)),
        ],
        compiler_params=pltpu.CompilerParams(collective_id=0),
    )(x, dest2d)


# device time: 8232 ns/iter; 2.0678x vs baseline; 1.6713x over previous
import os

import jax
import jax.numpy as jnp
from jax import lax
from jax.experimental import pallas as pl
from jax.experimental.pallas import tpu as pltpu

_NO_COMM = os.environ.get("KERNEL_NO_COMM") == "1"
_NO_COMPUTE = os.environ.get("KERNEL_NO_COMPUTE") == "1"

N_ROWS = 1024
N_COLS = 512
CHUNK = 128
MAX_CHUNKS = N_ROWS // CHUNK


def kernel(x, dest):
    dest2d = dest.reshape(1, N_ROWS)

    def body(x_ref, dest_ref, out_ref, send_buf, recv_buf, send_sems, recv_sems):
        my_x = lax.axis_index("x")
        my_y = lax.axis_index("y")
        my_z = lax.axis_index("z")
        partner = (1 - my_x, my_y, my_z)

        recv_buf[:, :] = jnp.zeros((N_ROWS, N_COLS), jnp.bfloat16)
        if not _NO_COMM:
            bsem = pltpu.get_barrier_semaphore()
            pl.semaphore_signal(
                bsem, inc=1, device_id=partner, device_id_type=pl.DeviceIdType.MESH
            )

        d = dest_ref[:, :]
        km = (d == my_x).astype(jnp.float32)
        sm = 1.0 - km

        i_idx = lax.broadcasted_iota(jnp.int32, (N_ROWS, N_ROWS), 0)
        j_idx = lax.broadcasted_iota(jnp.int32, (N_ROWS, N_ROWS), 1)
        diff = i_idx - j_idx
        upper = (diff < 0).astype(jnp.float32)
        kc = jnp.dot(km, upper, preferred_element_type=jnp.float32)
        pos = lax.broadcasted_iota(jnp.int32, (1, N_ROWS), 1).astype(jnp.float32)
        sc = pos - kc

        k_count = jnp.sum(km).astype(jnp.int32)
        s_count = N_ROWS - k_count

        off_keep = jnp.where(my_x == 0, 0, s_count)
        off_recv = jnp.where(my_x == 0, k_count, 0)

        kc_i = jnp.where(km > 0, off_keep + kc.astype(jnp.int32), -1)
        sc_i = jnp.where(sm > 0, sc.astype(jnp.int32), -1)

        xb = x_ref[:, :].astype(jnp.bfloat16)
        nch = (s_count + CHUNK - 1) // CHUNK
        if _NO_COMPUTE:
            nch = 4

        if not _NO_COMM:
            pl.semaphore_wait(bsem, 1)

        def chunk_rdma(j):
            return pltpu.make_async_remote_copy(
                src_ref=send_buf.at[pl.ds(j * CHUNK, CHUNK), :],
                dst_ref=recv_buf.at[pl.ds(j * CHUNK, CHUNK), :],
                send_sem=send_sems.at[j],
                recv_sem=recv_sems.at[j],
                device_id=partner,
                device_id_type=pl.DeviceIdType.MESH,
            )

        ii_c = lax.broadcasted_iota(jnp.int32, (CHUNK, N_ROWS), 0)
        for j in range(MAX_CHUNKS):
            @pl.when(j < nch)
            def _(j=j):
                if _NO_COMPUTE:
                    send_buf[pl.ds(j * CHUNK, CHUNK), :] = xb[
                        pl.ds(j * CHUNK, CHUNK), :
                    ]
                else:
                    ps_j = (ii_c == sc_i - j * CHUNK).astype(jnp.bfloat16)
                    send_buf[pl.ds(j * CHUNK, CHUNK), :] = jnp.dot(
                        ps_j, xb, preferred_element_type=jnp.float32
                    ).astype(jnp.bfloat16)
                if not _NO_COMM:
                    chunk_rdma(j).start()

        if not _NO_COMPUTE:
            pk = (i_idx == kc_i).astype(jnp.bfloat16)
            out_ref[:, :] = jnp.dot(
                pk, xb, preferred_element_type=jnp.float32
            ).astype(jnp.bfloat16)
        else:
            out_ref[:, :] = xb

        jj_c = lax.broadcasted_iota(jnp.int32, (N_ROWS, CHUNK), 1)
        i_col = lax.broadcasted_iota(jnp.int32, (N_ROWS, CHUNK), 0)
        for j in range(MAX_CHUNKS):
            @pl.when(j < nch)
            def _(j=j):
                if not _NO_COMM:
                    chunk_rdma(j).wait_recv()
                if not _NO_COMPUTE:
                    pr_j = (
                        i_col == off_recv + j * CHUNK + jj_c
                    ).astype(jnp.bfloat16)
                    out_ref[:, :] = out_ref[:, :] + jnp.dot(
                        pr_j,
                        recv_buf[pl.ds(j * CHUNK, CHUNK), :],
                        preferred_element_type=jnp.float32,
                    ).astype(jnp.bfloat16)

        if not _NO_COMM:
            for j in range(MAX_CHUNKS):
                @pl.when(j < nch)
                def _(j=j):
                    chunk_rdma(j).wait_send()

    return pl.pallas_call(
        body,
        out_shape=jax.ShapeDtypeStruct((N_ROWS, N_COLS), jnp.bfloat16),
        in_specs=[
            pl.BlockSpec(memory_space=pltpu.VMEM),
            pl.BlockSpec(memory_space=pltpu.VMEM),
        ],
        out_specs=pl.BlockSpec(memory_space=pltpu.VMEM),
        scratch_shapes=[
            pltpu.VMEM((N_ROWS, N_COLS), jnp.bfloat16),
            pltpu.VMEM((N_ROWS, N_COLS), jnp.bfloat16),
            pltpu.SemaphoreType.DMA((MAX_CHUNKS,)),
            pltpu.SemaphoreType.DMA((MAX_CHUNKS,)),
        ],
        compiler_params=pltpu.CompilerParams(
            collective_id=None if _NO_COMM else 0
        ),
    )(x, dest2d)
